# baseline (speedup 1.0000x reference)
import functools

import jax
import jax.numpy as jnp
from jax import lax
from jax.experimental import pallas as pl
from jax.experimental.pallas import tpu as pltpu
from jax.experimental.pallas import tpu_sc as plsc

_NC = 2
_NS = 16
_NW = _NC * _NS
_G = 128
_BPW = 8
_PSEQ = 56
_PDIM = 128


def _tc_pad(weight):
    v, dim = weight.shape
    blk = 8000

    def body(w_ref, o_ref):
        o_ref[:, :dim] = w_ref[...]
        o_ref[:, dim:] = jnp.zeros((blk, _PDIM - dim), jnp.float32)

    return pl.pallas_call(
        body,
        grid=(v // blk,),
        in_specs=[pl.BlockSpec((blk, dim), lambda i: (i, 0))],
        out_specs=pl.BlockSpec((blk, _PDIM), lambda i: (i, 0)),
        out_shape=jax.ShapeDtypeStruct((v, _PDIM), jnp.float32),
        compiler_params=pltpu.CompilerParams(
            dimension_semantics=("parallel",)),
    )(weight)


def _tc_slice(flat, batch, seq, dim):
    nb = 64

    def body(f_ref, o_ref):
        for j in range(nb):
            o_ref[j] = f_ref[pl.ds(j * _PSEQ, seq), pl.ds(0, dim)]

    return pl.pallas_call(
        body,
        grid=(batch // nb,),
        in_specs=[pl.BlockSpec((nb * _PSEQ, _PDIM), lambda i: (i, 0))],
        out_specs=pl.BlockSpec((nb, seq, dim), lambda i: (i, 0, 0)),
        out_shape=jax.ShapeDtypeStruct((batch, seq, dim), jnp.float32),
        compiler_params=pltpu.CompilerParams(
            dimension_semantics=("parallel",)),
    )(flat)


def _sc_gather(idx_flat, table):
    n = idx_flat.shape[0]
    ids_per_worker = n // _NW
    win_ids = _BPW * _PSEQ
    n_win = ids_per_worker // win_ids
    chunks = []
    off = 0
    while off < win_ids:
        c = min(_G, win_ids - off)
        chunks.append((off, c))
        off += c
    mesh = plsc.VectorSubcoreMesh(core_axis_name="c", subcore_axis_name="s")

    @functools.partial(
        pl.kernel,
        mesh=mesh,
        out_type=jax.ShapeDtypeStruct((n, _PDIM), jnp.float32),
        scratch_types=[
            pltpu.VMEM((2, win_ids), jnp.int32),
            pltpu.VMEM((2, win_ids, _PDIM), jnp.float32),
            pltpu.SemaphoreType.DMA((2,)),
            pltpu.SemaphoreType.DMA((2,)),
            pltpu.SemaphoreType.DMA((2,)),
        ],
        compiler_params=pltpu.CompilerParams(use_tc_tiling_on_sc=False),
    )
    def gather_kernel(idx_hbm, table_hbm, out_hbm, idx_v, rows_v, sem_i,
                      sem_g, sem_o):
        wid = lax.axis_index("s") * _NC + lax.axis_index("c")
        id_base = wid * ids_per_worker

        def issue_idx(w, b):
            pltpu.async_copy(
                idx_hbm.at[pl.ds(id_base + w * win_ids, win_ids)],
                idx_v.at[b], sem_i.at[b])

        def wait_idx(b):
            pltpu.make_async_copy(
                idx_hbm.at[pl.ds(0, win_ids)], idx_v.at[b], sem_i.at[b]).wait()

        def issue_gathers(b):
            for (o, c) in chunks:
                pltpu.async_copy(
                    table_hbm.at[idx_v.at[b, pl.ds(o, c)]],
                    rows_v.at[b, pl.ds(o, c)], sem_g.at[b])

        def wait_gathers(b):
            for (o, c) in chunks:
                pltpu.make_async_copy(
                    table_hbm.at[pl.ds(0, c)],
                    rows_v.at[b, pl.ds(o, c)], sem_g.at[b]).wait()

        def issue_out(w, b):
            pltpu.async_copy(
                rows_v.at[b],
                out_hbm.at[pl.ds(id_base + w * win_ids, win_ids)],
                sem_o.at[b])

        def wait_out(b):
            pltpu.make_async_copy(
                rows_v.at[b], out_hbm.at[pl.ds(0, win_ids)],
                sem_o.at[b]).wait()

        issue_idx(0, 0)
        issue_idx(1, 1)
        wait_idx(0)
        issue_gathers(0)

        @pl.loop(0, n_win - 2, step=2)
        def _(g):
            wait_idx(1)

            @pl.when(g > 0)
            def _():
                wait_out(1)

            issue_gathers(1)
            wait_gathers(0)
            issue_out(g, 0)
            issue_idx(g + 2, 0)

            wait_idx(0)
            wait_out(0)
            issue_gathers(0)
            wait_gathers(1)
            issue_out(g + 1, 1)
            issue_idx(g + 3, 1)

        wait_idx(1)
        wait_out(1)
        issue_gathers(1)
        wait_gathers(0)
        issue_out(n_win - 2, 0)
        wait_gathers(1)
        issue_out(n_win - 1, 1)
        wait_out(0)
        wait_out(1)

    return gather_kernel(idx_flat, table)


def kernel(x, weight):
    b, s = x.shape
    dim = weight.shape[1]
    xpad = jnp.concatenate(
        [x, jnp.broadcast_to(x[:, :1], (b, _PSEQ - s))], axis=1)
    idx_flat = xpad.reshape(b * _PSEQ).astype(jnp.int32)
    table = _tc_pad(weight)
    flat = _sc_gather(idx_flat, table)
    return _tc_slice(flat, b, s, dim)


# speedup vs baseline: 1.5628x; 1.5628x over previous
import functools

import jax
import jax.numpy as jnp
from jax import lax
from jax.experimental import pallas as pl
from jax.experimental.pallas import tpu as pltpu
from jax.experimental.pallas import tpu_sc as plsc

_NC = 2
_NS = 16
_NW = _NC * _NS
_G = 128
_BPW = 8
_PSEQ = 56
_PDIM = 128


def _tc_pad(weight):
    v, dim = weight.shape
    blk = 8000

    def body(w_ref, o_ref):
        o_ref[:, :dim] = w_ref[...]
        o_ref[:, dim:] = jnp.zeros((blk, _PDIM - dim), jnp.float32)

    return pl.pallas_call(
        body,
        grid=(v // blk,),
        in_specs=[pl.BlockSpec((blk, dim), lambda i: (i, 0))],
        out_specs=pl.BlockSpec((blk, _PDIM), lambda i: (i, 0)),
        out_shape=jax.ShapeDtypeStruct((v, _PDIM), jnp.float32),
        compiler_params=pltpu.CompilerParams(
            dimension_semantics=("parallel",)),
    )(weight)


def _tc_slice(flat, batch, seq, dim):
    nb = 64

    def body(f_ref, o_ref):
        for j in range(nb):
            o_ref[j] = f_ref[pl.ds(j * _PSEQ, seq), pl.ds(0, dim)]

    return pl.pallas_call(
        body,
        grid=(batch // nb,),
        in_specs=[pl.BlockSpec((nb * _PSEQ, _PDIM), lambda i: (i, 0))],
        out_specs=pl.BlockSpec((nb, seq, dim), lambda i: (i, 0, 0)),
        out_shape=jax.ShapeDtypeStruct((batch, seq, dim), jnp.float32),
        compiler_params=pltpu.CompilerParams(
            dimension_semantics=("parallel",)),
    )(flat)


def _sc_gather(idx_flat, table):
    n = idx_flat.shape[0]
    ids_per_worker = n // _NW
    win_ids = _BPW * _PSEQ
    n_win = ids_per_worker // win_ids
    chunks = []
    off = 0
    while off < win_ids:
        c = min(_G, win_ids - off)
        chunks.append((off, c))
        off += c
    mesh = plsc.VectorSubcoreMesh(core_axis_name="c", subcore_axis_name="s")

    @functools.partial(
        pl.kernel,
        mesh=mesh,
        out_type=jax.ShapeDtypeStruct((n, _PDIM), jnp.float32),
        scratch_types=[
            pltpu.VMEM((2, win_ids), jnp.int32),
            pltpu.VMEM((2, win_ids, _PDIM), jnp.float32),
            pltpu.SemaphoreType.DMA((2,)),
            pltpu.SemaphoreType.DMA((2,)),
            pltpu.SemaphoreType.DMA((2,)),
        ],
        compiler_params=pltpu.CompilerParams(use_tc_tiling_on_sc=False),
    )
    def gather_kernel(idx_hbm, table_hbm, out_hbm, idx_v, rows_v, sem_i,
                      sem_g, sem_o):
        wid = lax.axis_index("s") * _NC + lax.axis_index("c")
        id_base = wid * ids_per_worker

        def issue_idx(w, b):
            pltpu.async_copy(
                idx_hbm.at[pl.ds(id_base + w * win_ids, win_ids)],
                idx_v.at[b], sem_i.at[b])

        def wait_idx(b):
            pltpu.make_async_copy(
                idx_hbm.at[pl.ds(0, win_ids)], idx_v.at[b], sem_i.at[b]).wait()

        def issue_gathers(b):
            for (o, c) in chunks:
                pltpu.async_copy(
                    table_hbm.at[idx_v.at[b, pl.ds(o, c)]],
                    rows_v.at[b, pl.ds(o, c)], sem_g.at[b])

        def wait_gathers(b):
            for (o, c) in chunks:
                pltpu.make_async_copy(
                    table_hbm.at[pl.ds(0, c)],
                    rows_v.at[b, pl.ds(o, c)], sem_g.at[b]).wait()

        def issue_out(w, b):
            pltpu.async_copy(
                rows_v.at[b],
                out_hbm.at[pl.ds(id_base + w * win_ids, win_ids)],
                sem_o.at[b])

        def wait_out(b):
            pltpu.make_async_copy(
                rows_v.at[b], out_hbm.at[pl.ds(0, win_ids)],
                sem_o.at[b]).wait()

        issue_idx(0, 0)
        issue_idx(1, 1)
        wait_idx(0)
        issue_gathers(0)

        @pl.loop(0, n_win - 2, step=2)
        def _(g):
            wait_idx(1)

            @pl.when(g > 0)
            def _():
                wait_out(1)

            issue_gathers(1)
            wait_gathers(0)
            issue_out(g, 0)
            issue_idx(g + 2, 0)

            wait_idx(0)
            wait_out(0)
            issue_gathers(0)
            wait_gathers(1)
            issue_out(g + 1, 1)
            issue_idx(g + 3, 1)

        wait_idx(1)
        wait_out(1)
        issue_gathers(1)
        wait_gathers(0)
        issue_out(n_win - 2, 0)
        wait_gathers(1)
        issue_out(n_win - 1, 1)
        wait_out(0)
        wait_out(1)

    return gather_kernel(idx_flat, table)


def kernel(x, weight):
    b, s = x.shape
    dim = weight.shape[1]
    xpad = jnp.concatenate(
        [x, jnp.broadcast_to(x[:, :1], (b, _PSEQ - s))], axis=1)
    idx_flat = xpad.reshape(b * _PSEQ).astype(jnp.int32)
    table = jnp.pad(weight, ((0, 0), (0, _PDIM - dim)))
    flat = _sc_gather(idx_flat, table)
    return flat.reshape(b, _PSEQ, _PDIM)[:, :s, :dim]
